# baseline (speedup 1.0000x reference)
import functools

import jax
import jax.numpy as jnp
from jax import lax
from jax.experimental import pallas as pl
from jax.experimental.pallas import tpu as pltpu
from jax.experimental.pallas import tpu_sc as plsc

VOCAB = 1000000
DIM = 16
BATCH = 16384
CTX = 20

NC = 2
NS = 16
NW = NC * NS
B_PER_W = BATCH // NW
CB = 128
NCHUNK = B_PER_W // CB
PAIRS = CB * CTX


def _body(focus_hbm, ctx_hbm, win_hbm, wout_hbm, out_hbm,
          idx_f, idx_c, frows, crows, out_v, sem):
    wid = lax.axis_index("s") * NC + lax.axis_index("c")
    iota16 = lax.iota(jnp.int32, 16)

    def chunk_body(c, carry):
        brow = wid * NCHUNK + c
        pltpu.sync_copy(focus_hbm.at[pl.ds(brow * CB, CB)], idx_f)
        pltpu.sync_copy(ctx_hbm.at[pl.ds(brow * PAIRS, PAIRS)], idx_c)
        copies = [pltpu.async_copy(win_hbm.at[idx_f], frows, sem)]
        for j in range(CTX):
            copies.append(pltpu.async_copy(
                wout_hbm.at[idx_c.at[pl.ds(j * CB, CB)]],
                crows.at[pl.ds(j * CB, CB)], sem))
        for cp in copies:
            cp.wait()

        def g_body(g, carry2):
            bvec = g * 16 + iota16
            fcols = [plsc.load_gather(frows, [bvec, jnp.full((16,), d, jnp.int32)])
                     for d in range(DIM)]
            base = bvec * CTX

            def l_body(l, carry3):
                crow = base + l
                acc = jnp.zeros((16,), jnp.float32)
                for d in range(DIM):
                    cv = plsc.load_gather(
                        crows, [crow, jnp.full((16,), d, jnp.int32)])
                    acc = acc + cv * fcols[d]
                plsc.store_scatter(out_v, [crow], acc)
                return carry3

            lax.fori_loop(0, CTX, l_body, 0)
            return carry2

        lax.fori_loop(0, CB // 16, g_body, 0)
        pltpu.sync_copy(out_v, out_hbm.at[pl.ds(brow * PAIRS, PAIRS)])
        return carry

    lax.fori_loop(0, NCHUNK, chunk_body, 0)


def kernel(focus_item_batch, context_items_batch, W_in, W_out):
    focus2 = focus_item_batch.reshape(BATCH).astype(jnp.int32)
    ctx2 = context_items_batch.reshape(BATCH * CTX).astype(jnp.int32)
    run = pl.kernel(
        _body,
        out_type=jax.ShapeDtypeStruct((BATCH * CTX,), jnp.float32),
        mesh=plsc.VectorSubcoreMesh(core_axis_name="c", subcore_axis_name="s"),
        compiler_params=pltpu.CompilerParams(
            needs_layout_passes=False, use_tc_tiling_on_sc=False),
        scratch_types=[
            pltpu.VMEM((CB,), jnp.int32),
            pltpu.VMEM((PAIRS,), jnp.int32),
            pltpu.VMEM((CB, DIM), jnp.float32),
            pltpu.VMEM((PAIRS, DIM), jnp.float32),
            pltpu.VMEM((PAIRS,), jnp.float32),
            pltpu.SemaphoreType.DMA,
        ],
    )
    out = run(focus2, ctx2, W_in, W_out)
    return out.reshape(BATCH, CTX)
